# Initial kernel scaffold; baseline (speedup 1.0000x reference)
import jax
import jax.numpy as jnp
from jax.experimental import pallas as pl


def kernel(x, entity_emb, relation_emb):
    raise NotImplementedError("write your pallas kernel here")


# speedup vs baseline: 1.7705x; 1.7705x over previous
import functools

import jax
import jax.numpy as jnp
from jax import lax
from jax.experimental import pallas as pl
from jax.experimental.pallas import tpu as pltpu
from jax.experimental.pallas import tpu_sc as plsc

BATCH = 16384
EMB = 128
LANES = 16
NUM_CORES = 2
NUM_SUBCORES = 16
NUM_WORKERS = NUM_CORES * NUM_SUBCORES
BPW = BATCH // NUM_WORKERS
CHUNK = 128
NCHUNK = BPW // CHUNK
DCHUNKS = EMB // LANES


def _score_body(e1i_hbm, ri_hbm, e2i_hbm, ent_hbm, rel_hbm, out_hbm,
                idx1, idxr, idx2, rows1, rowsr, rows2, accs, out_v, sem):
  wid = lax.axis_index("s") * NUM_CORES + lax.axis_index("c")
  base = wid * BPW

  for ck in range(NCHUNK):
    off = base + ck * CHUNK
    pltpu.sync_copy(e1i_hbm.at[pl.ds(off, CHUNK)], idx1)
    pltpu.sync_copy(ri_hbm.at[pl.ds(off, CHUNK)], idxr)
    pltpu.sync_copy(e2i_hbm.at[pl.ds(off, CHUNK)], idx2)
    g1 = pltpu.async_copy(ent_hbm.at[idx1], rows1, sem)
    g2 = pltpu.async_copy(rel_hbm.at[idxr], rowsr, sem)
    g3 = pltpu.async_copy(ent_hbm.at[idx2], rows2, sem)
    g1.wait()
    g2.wait()
    g3.wait()

    def group(g, carry):
      for i in range(LANES):
        row = g * LANES + i
        acc = (rows1[row, pl.ds(0, LANES)]
               * rowsr[row, pl.ds(0, LANES)]
               * rows2[row, pl.ds(0, LANES)])
        for j in range(1, DCHUNKS):
          acc = acc + (rows1[row, pl.ds(j * LANES, LANES)]
                       * rowsr[row, pl.ds(j * LANES, LANES)]
                       * rows2[row, pl.ds(j * LANES, LANES)])
        accs[pl.ds(i * LANES, LANES)] = acc
      iota = lax.iota(jnp.int32, LANES)
      rowbase = iota * LANES
      ssum = plsc.load_gather(accs, [rowbase + iota])
      for d in range(1, LANES):
        col = jnp.bitwise_and(iota + d, LANES - 1)
        ssum = ssum + plsc.load_gather(accs, [rowbase + col])
      out_v[pl.ds(g * LANES, LANES)] = ssum
      return carry

    lax.fori_loop(0, CHUNK // LANES, group, 0)
    pltpu.sync_copy(out_v, out_hbm.at[pl.ds(off, CHUNK)])


@functools.partial(
    pl.kernel,
    out_type=jax.ShapeDtypeStruct((BATCH,), jnp.float32),
    mesh=plsc.VectorSubcoreMesh(core_axis_name="c", subcore_axis_name="s"),
    scratch_types=[
        pltpu.VMEM((CHUNK,), jnp.int32),
        pltpu.VMEM((CHUNK,), jnp.int32),
        pltpu.VMEM((CHUNK,), jnp.int32),
        pltpu.VMEM((CHUNK, EMB), jnp.float32),
        pltpu.VMEM((CHUNK, EMB), jnp.float32),
        pltpu.VMEM((CHUNK, EMB), jnp.float32),
        pltpu.VMEM((LANES * LANES,), jnp.float32),
        pltpu.VMEM((CHUNK,), jnp.float32),
        pltpu.SemaphoreType.DMA,
    ],
    compiler_params=pltpu.CompilerParams(needs_layout_passes=False),
)
def _score_kernel(e1i, ri, e2i, ent, rel, out,
                  idx1, idxr, idx2, rows1, rowsr, rows2, accs, out_v, sem):
  _score_body(e1i, ri, e2i, ent, rel, out,
              idx1, idxr, idx2, rows1, rowsr, rows2, accs, out_v, sem)


@jax.jit
def kernel(x, entity_emb, relation_emb):
  e1i = x[:, 0]
  ri = x[:, 1]
  e2i = x[:, 2]
  return _score_kernel(e1i, ri, e2i, entity_emb, relation_emb)


# speedup vs baseline: 2.2149x; 1.2510x over previous
import functools

import jax
import jax.numpy as jnp
from jax import lax
from jax.experimental import pallas as pl
from jax.experimental.pallas import tpu as pltpu
from jax.experimental.pallas import tpu_sc as plsc

BATCH = 16384
EMB = 128
LANES = 16
NUM_CORES = 2
NUM_SUBCORES = 16
NUM_WORKERS = NUM_CORES * NUM_SUBCORES
BPW = BATCH // NUM_WORKERS
CHUNK = 128
NCHUNK = BPW // CHUNK
DCHUNKS = EMB // LANES
GROUPS = CHUNK // LANES


def _compute_chunk(rows1, rowsr, rows2, accs, out_v, ck):
  def group(g, carry):
    for i in range(LANES):
      row = g * LANES + i
      acc = (rows1[row, pl.ds(0, LANES)]
             * rowsr[row, pl.ds(0, LANES)]
             * rows2[row, pl.ds(0, LANES)])
      for j in range(1, DCHUNKS):
        acc = acc + (rows1[row, pl.ds(j * LANES, LANES)]
                     * rowsr[row, pl.ds(j * LANES, LANES)]
                     * rows2[row, pl.ds(j * LANES, LANES)])
      accs[pl.ds(i * LANES, LANES)] = acc
    iota = lax.iota(jnp.int32, LANES)
    rowbase = iota * LANES
    ssum = plsc.load_gather(accs, [rowbase + iota])
    for d in range(1, LANES):
      col = jnp.bitwise_and(iota + d, LANES - 1)
      ssum = ssum + plsc.load_gather(accs, [rowbase + col])
    out_v[pl.ds(ck * CHUNK + g * LANES, LANES)] = ssum
    return carry

  lax.fori_loop(0, GROUPS, group, 0)


def _score_body(e1i_hbm, ri_hbm, e2i_hbm, ent_hbm, rel_hbm, out_hbm,
                idx1, idxr, idx2,
                rows1a, rowsra, rows2a, rows1b, rowsrb, rows2b,
                accs, out_v, sem0, sem1):
  wid = lax.axis_index("s") * NUM_CORES + lax.axis_index("c")
  base = wid * BPW

  pltpu.sync_copy(e1i_hbm.at[pl.ds(wid * NCHUNK, NCHUNK)], idx1)
  pltpu.sync_copy(ri_hbm.at[pl.ds(wid * NCHUNK, NCHUNK)], idxr)
  pltpu.sync_copy(e2i_hbm.at[pl.ds(wid * NCHUNK, NCHUNK)], idx2)

  rows1 = (rows1a, rows1b)
  rowsr = (rowsra, rowsrb)
  rows2 = (rows2a, rows2b)
  sems = (sem0, sem1)

  def fire(ck):
    buf = ck % 2
    return (
        pltpu.async_copy(ent_hbm.at[idx1.at[ck]], rows1[buf], sems[buf]),
        pltpu.async_copy(rel_hbm.at[idxr.at[ck]], rowsr[buf], sems[buf]),
        pltpu.async_copy(ent_hbm.at[idx2.at[ck]], rows2[buf], sems[buf]),
    )

  pending = fire(0)
  for ck in range(NCHUNK):
    buf = ck % 2
    cur = pending
    if ck + 1 < NCHUNK:
      pending = fire(ck + 1)
    for h in cur:
      h.wait()
    _compute_chunk(rows1[buf], rowsr[buf], rows2[buf], accs, out_v, ck)

  pltpu.sync_copy(out_v, out_hbm.at[pl.ds(base, BPW)])


@functools.partial(
    pl.kernel,
    out_type=jax.ShapeDtypeStruct((BATCH,), jnp.float32),
    mesh=plsc.VectorSubcoreMesh(core_axis_name="c", subcore_axis_name="s"),
    scratch_types=[
        pltpu.VMEM((NCHUNK, CHUNK), jnp.int32),
        pltpu.VMEM((NCHUNK, CHUNK), jnp.int32),
        pltpu.VMEM((NCHUNK, CHUNK), jnp.int32),
        pltpu.VMEM((CHUNK, EMB), jnp.float32),
        pltpu.VMEM((CHUNK, EMB), jnp.float32),
        pltpu.VMEM((CHUNK, EMB), jnp.float32),
        pltpu.VMEM((CHUNK, EMB), jnp.float32),
        pltpu.VMEM((CHUNK, EMB), jnp.float32),
        pltpu.VMEM((CHUNK, EMB), jnp.float32),
        pltpu.VMEM((LANES * LANES,), jnp.float32),
        pltpu.VMEM((BPW,), jnp.float32),
        pltpu.SemaphoreType.DMA,
        pltpu.SemaphoreType.DMA,
    ],
    compiler_params=pltpu.CompilerParams(needs_layout_passes=False),
)
def _score_kernel(e1i, ri, e2i, ent, rel, out,
                  idx1, idxr, idx2,
                  rows1a, rowsra, rows2a, rows1b, rowsrb, rows2b,
                  accs, out_v, sem0, sem1):
  _score_body(e1i, ri, e2i, ent, rel, out,
              idx1, idxr, idx2,
              rows1a, rowsra, rows2a, rows1b, rowsrb, rows2b,
              accs, out_v, sem0, sem1)


@jax.jit
def kernel(x, entity_emb, relation_emb):
  e1i = x[:, 0].reshape(NUM_WORKERS * NCHUNK, CHUNK)
  ri = x[:, 1].reshape(NUM_WORKERS * NCHUNK, CHUNK)
  e2i = x[:, 2].reshape(NUM_WORKERS * NCHUNK, CHUNK)
  return _score_kernel(e1i, ri, e2i, entity_emb, relation_emb)
